# baseline (speedup 1.0000x reference)
import functools

import jax
import jax.numpy as jnp
from jax import lax
from jax.experimental import pallas as pl
from jax.experimental.pallas import tpu as pltpu
from jax.experimental.pallas import tpu_sc as plsc

N = 10000
D = 128
E = 320000
NC = 2
NS = 16
C = 128
K = 80
E_PAD = NC * NS * C * K
N_PAD = 10240
ROWS_PER_TILE = N_PAD // NS


def _sc_phase_body(tbl_hbm, pidx_hbm, w_hbm, out_hbm,
                   pidx_v, w_v, rows0, rows1, gbuf, sbuf0, sbuf1, acc,
                   g0, g1, s0, s1):
    c = lax.axis_index("c")
    s = lax.axis_index("s")
    wid = c * NS + s
    row0 = pl.multiple_of(s * ROWS_PER_TILE, C)
    idx0 = pl.multiple_of(wid * K, 16)

    RW = (rows0, rows1)
    SBUF = (sbuf0, sbuf1)
    GS = (g0, g1)
    SS = (s0, s1)

    def zero_body(i, carry):
        for q in range(D // 16):
            rows0[i, pl.ds(q * 16, 16)] = jnp.zeros((16,), jnp.float32)
        return carry

    lax.fori_loop(0, C, zero_body, 0)
    for p in range(ROWS_PER_TILE // C):
        pltpu.sync_copy(rows0, acc.at[pl.ds(row0 + p * C, C)])
    plsc.subcore_barrier()

    pltpu.sync_copy(pidx_hbm.at[pl.ds(idx0, K)], pidx_v)
    pltpu.sync_copy(w_hbm.at[pl.ds(pl.multiple_of(wid * (K // 2), 8), K // 2)],
                    w_v)

    def unpack_idx(j, sb):
        for g in range(C // 16):
            v = pidx_v[j, pl.ds(g * 16, 16)]
            gbuf[pl.ds(g * 16, 16)] = jnp.bitwise_and(v, jnp.int32(0xFFFF))
            sb[pl.ds(g * 16, 16)] = jnp.right_shift(v, jnp.int32(16))

    def scale(rows, j):
        def grp_body(g, carry2):
            wu32 = w_v[lax.div(j, 2),
                       pl.ds(lax.rem(j, 2) * (C // 2) + g * 16, 16)]
            evens = lax.bitcast_convert_type(jnp.left_shift(wu32, 16),
                                             jnp.float32)
            odds = lax.bitcast_convert_type(
                jnp.bitwise_and(wu32, jnp.int32(-65536)), jnp.float32)
            for e in range(32):
                vec = evens if e % 2 == 0 else odds
                wsp = vec.at[jnp.full((16,), e // 2, jnp.int32)].get(
                    mode="promise_in_bounds")
                r = g * 32 + e
                vals = [rows[r, pl.ds(q * 16, 16)] for q in range(D // 16)]
                prods = [v * wsp for v in vals]
                for q in range(D // 16):
                    rows[r, pl.ds(q * 16, 16)] = prods[q]
            return carry2

        lax.fori_loop(0, C // 32, grp_body, 0)

    def chunk_pair(t, carry):
        for pr in range(2):
            j = 2 * t + pr

            @pl.when(t > 0)
            def _():
                pltpu.make_async_copy(RW[pr], acc.at[SBUF[pr]],
                                      SS[pr]).wait()

            unpack_idx(j, SBUF[pr])
            pltpu.async_copy(tbl_hbm.at[gbuf], RW[pr], GS[pr]).wait()
            scale(RW[pr], j)
            pltpu.async_copy(RW[pr], acc.at[SBUF[pr]], SS[pr], add=True)
        return carry

    lax.fori_loop(0, K // 2, chunk_pair, 0)
    pltpu.make_async_copy(RW[0], acc.at[SBUF[0]], SS[0]).wait()
    pltpu.make_async_copy(RW[1], acc.at[SBUF[1]], SS[1]).wait()
    plsc.subcore_barrier()

    pltpu.sync_copy(acc.at[pl.ds(row0, ROWS_PER_TILE)],
                    out_hbm.at[c].at[pl.ds(row0, ROWS_PER_TILE)])


_sc_phase = functools.partial(
    pl.kernel,
    mesh=plsc.VectorSubcoreMesh(core_axis_name="c", subcore_axis_name="s"),
    out_type=jax.ShapeDtypeStruct((NC, N_PAD, D), jnp.float32),
    scratch_types=[
        pltpu.VMEM((K, C), jnp.int32),
        pltpu.VMEM((K // 2, C), jnp.int32),
        pltpu.VMEM((C, D), jnp.float32),
        pltpu.VMEM((C, D), jnp.float32),
        pltpu.VMEM((C,), jnp.int32),
        pltpu.VMEM((C,), jnp.int32),
        pltpu.VMEM((C,), jnp.int32),
        pltpu.VMEM_SHARED((N_PAD, D), jnp.float32),
        pltpu.SemaphoreType.DMA,
        pltpu.SemaphoreType.DMA,
        pltpu.SemaphoreType.DMA,
        pltpu.SemaphoreType.DMA,
    ],
)(_sc_phase_body)


def _mm_kernel(a_ref, w_ref, o_ref):
    o_ref[...] = jnp.dot(a_ref[...], w_ref[...],
                         preferred_element_type=jnp.float32,
                         precision=lax.Precision.HIGHEST)


def _mm2_kernel(p_ref, w_ref, o_ref):
    a = p_ref[0] + p_ref[1]
    o_ref[...] = jnp.dot(a, w_ref[...], preferred_element_type=jnp.float32,
                         precision=lax.Precision.HIGHEST)


def _lin_kernel(x_ref, p_ref, wt_ref, wb_ref, b_ref, o_ref):
    acc = jnp.dot(x_ref[...], wt_ref[...], preferred_element_type=jnp.float32,
                  precision=lax.Precision.HIGHEST)
    acc += jnp.dot(p_ref[0] + p_ref[1], wb_ref[...],
                   preferred_element_type=jnp.float32,
                   precision=lax.Precision.HIGHEST)
    o_ref[...] = acc + b_ref[...]


_B = 1000
_G = N // _B


def _mm(x, w):
    return pl.pallas_call(
        _mm_kernel,
        grid=(_G,),
        in_specs=[pl.BlockSpec((_B, D), lambda i: (i, 0)),
                  pl.BlockSpec((D, D), lambda i: (0, 0))],
        out_specs=pl.BlockSpec((_B, D), lambda i: (i, 0)),
        out_shape=jax.ShapeDtypeStruct((N, D), jnp.float32),
    )(x, w)


def _mm2(parts, w):
    return pl.pallas_call(
        _mm2_kernel,
        grid=(_G,),
        in_specs=[pl.BlockSpec((NC, _B, D), lambda i: (0, i, 0)),
                  pl.BlockSpec((D, D), lambda i: (0, 0))],
        out_specs=pl.BlockSpec((_B, D), lambda i: (i, 0)),
        out_shape=jax.ShapeDtypeStruct((N, D), jnp.float32),
    )(parts, w)


def _lin(x, parts, wt, wb, b):
    return pl.pallas_call(
        _lin_kernel,
        grid=(_G,),
        in_specs=[pl.BlockSpec((_B, D), lambda i: (i, 0)),
                  pl.BlockSpec((NC, _B, D), lambda i: (0, i, 0)),
                  pl.BlockSpec((D, D), lambda i: (0, 0)),
                  pl.BlockSpec((D, D), lambda i: (0, 0)),
                  pl.BlockSpec((1, D), lambda i: (0, 0))],
        out_specs=pl.BlockSpec((_B, D), lambda i: (i, 0)),
        out_shape=jax.ShapeDtypeStruct((N, D), jnp.float32),
    )(x, parts, wt, wb, b)


def kernel(x, edge_index, edge_weight, W_v2e, W_e2v, W_lin, b_lin):
    src = edge_index[0]
    dst = edge_index[1]
    pad = E_PAD - E
    pad_idx = (jnp.arange(pad, dtype=jnp.int32) * 97) % N
    shp = (E_PAD // C, C)
    srcp = jnp.concatenate([src, pad_idx])
    dstp = jnp.concatenate([dst, pad_idx])
    fwd = (srcp | (dstp << 16)).reshape(shp)
    rev = (dstp | (srcp << 16)).reshape(shp)
    wp = jnp.concatenate([edge_weight, jnp.zeros((pad,), jnp.float32)])
    wbits = jax.lax.bitcast_convert_type(wp, jnp.uint32)
    wr = (wbits + jnp.uint32(0x8000)) & jnp.uint32(0xFFFF0000)
    wpk = jax.lax.bitcast_convert_type(
        (wr[0::2] >> jnp.uint32(16)) | wr[1::2],
        jnp.int32).reshape(E_PAD // (2 * C), C)

    h1 = _mm(x, W_v2e)
    xe_parts = _sc_phase(h1, fwd, wpk)
    h2 = _mm2(xe_parts, W_e2v)
    xv_parts = _sc_phase(h2, rev, wpk)
    return _lin(x, xv_parts, W_lin[:D], W_lin[D:], b_lin[None, :])


# speedup vs baseline: 1.0547x; 1.0547x over previous
import functools

import jax
import jax.numpy as jnp
from jax import lax
from jax.experimental import pallas as pl
from jax.experimental.pallas import tpu as pltpu
from jax.experimental.pallas import tpu_sc as plsc

N = 10000
D = 128
E = 320000
NC = 2
NS = 16
C = 128
K = 80
E_PAD = NC * NS * C * K
N_PAD = 10240
ROWS_PER_TILE = N_PAD // NS


NB = K // 8
SB = K // 16


def _sc_phase_body(tbl_hbm, gidx_hbm, sidx_hbm, w_hbm, out_hbm,
                   gi0, gi1, si0, si1, wb0, wb1, rows0, rows1, acc,
                   i0, i1, g0, g1, s0, s1):
    c = lax.axis_index("c")
    s = lax.axis_index("s")
    wid = c * NS + s
    row0 = pl.multiple_of(s * ROWS_PER_TILE, C)
    blk0 = wid * NB

    GI = (gi0, gi1)
    SI = (si0, si1)
    WB = (wb0, wb1)
    RW = (rows0, rows1)
    IS = (i0, i1)
    GS = (g0, g1)
    SS = (s0, s1)

    def zero_body(i, carry):
        for q in range(D // 16):
            rows0[i, pl.ds(q * 16, 16)] = jnp.zeros((16,), jnp.float32)
        return carry

    lax.fori_loop(0, C, zero_body, 0)
    for p in range(ROWS_PER_TILE // C):
        pltpu.sync_copy(rows0, acc.at[pl.ds(row0 + p * C, C)])
    plsc.subcore_barrier()

    def load_blockset(blk, p):
        pltpu.async_copy(gidx_hbm.at[blk], GI[p], IS[p])
        pltpu.async_copy(sidx_hbm.at[blk], SI[p], IS[p])
        pltpu.async_copy(w_hbm.at[blk], WB[p], IS[p])

    def wait_blockset(blk, p):
        pltpu.make_async_copy(gidx_hbm.at[blk], GI[p], IS[p]).wait()
        pltpu.make_async_copy(sidx_hbm.at[blk], SI[p], IS[p]).wait()
        pltpu.make_async_copy(w_hbm.at[blk], WB[p], IS[p]).wait()

    def scale(rows, wrow):
        def grp_body(g, carry2):
            w16 = wrow[pl.ds(lax.div(g, 2) * 16, 16)]
            for e in range(8):
                lane = lax.rem(g, 2) * 8 + e
                wsp = w16.at[jnp.full((16,), lane, jnp.int32)].get(
                    mode="promise_in_bounds")
                r = g * 8 + e
                vals = [rows[r, pl.ds(q * 16, 16)] for q in range(D // 16)]
                prods = [v * wsp for v in vals]
                for q in range(D // 16):
                    rows[r, pl.ds(q * 16, 16)] = prods[q]
            return carry2

        lax.fori_loop(0, C // 8, grp_body, 0)

    load_blockset(blk0, 0)
    wait_blockset(blk0, 0)

    def superblock(t, carry):
        for u in range(16):
            pr = u % 2
            p = u // 8

            if u < 2:
                @pl.when(t > 0)
                def _():
                    pltpu.make_async_copy(RW[pr], acc.at[SI[1].at[6 + u]],
                                          SS[pr]).wait()
            else:
                pltpu.make_async_copy(RW[pr],
                                      acc.at[SI[(u - 2) // 8].at[(u - 2) % 8]],
                                      SS[pr]).wait()

            if u == 1:
                load_blockset(blk0 + 2 * t + 1, 1)
            if u == 9:
                @pl.when(t < SB - 1)
                def _():
                    load_blockset(blk0 + 2 * t + 2, 0)
            if u == 7:
                wait_blockset(blk0 + 2 * t + 1, 1)
            if u == 15:
                @pl.when(t < SB - 1)
                def _():
                    wait_blockset(blk0 + 2 * t + 2, 0)

            pltpu.async_copy(tbl_hbm.at[GI[p].at[u % 8]], RW[pr],
                             GS[pr]).wait()
            scale(RW[pr], WB[p].at[u % 8])
            pltpu.async_copy(RW[pr], acc.at[SI[p].at[u % 8]], SS[pr],
                             add=True)
        return carry

    lax.fori_loop(0, SB, superblock, 0)
    pltpu.make_async_copy(RW[0], acc.at[SI[1].at[6]], SS[0]).wait()
    pltpu.make_async_copy(RW[1], acc.at[SI[1].at[7]], SS[1]).wait()
    plsc.subcore_barrier()

    pltpu.sync_copy(acc.at[pl.ds(row0, ROWS_PER_TILE)],
                    out_hbm.at[c].at[pl.ds(row0, ROWS_PER_TILE)])


_sc_phase = functools.partial(
    pl.kernel,
    mesh=plsc.VectorSubcoreMesh(core_axis_name="c", subcore_axis_name="s"),
    out_type=jax.ShapeDtypeStruct((NC, N_PAD, D), jnp.float32),
    scratch_types=[
        pltpu.VMEM((8, C), jnp.int32),
        pltpu.VMEM((8, C), jnp.int32),
        pltpu.VMEM((8, C), jnp.int32),
        pltpu.VMEM((8, C), jnp.int32),
        pltpu.VMEM((8, C), jnp.float32),
        pltpu.VMEM((8, C), jnp.float32),
        pltpu.VMEM((C, D), jnp.float32),
        pltpu.VMEM((C, D), jnp.float32),
        pltpu.VMEM_SHARED((N_PAD, D), jnp.float32),
        pltpu.SemaphoreType.DMA,
        pltpu.SemaphoreType.DMA,
        pltpu.SemaphoreType.DMA,
        pltpu.SemaphoreType.DMA,
        pltpu.SemaphoreType.DMA,
        pltpu.SemaphoreType.DMA,
    ],
)(_sc_phase_body)


def _mm_kernel(a_ref, w_ref, o_ref):
    o_ref[...] = jnp.dot(a_ref[...], w_ref[...],
                         preferred_element_type=jnp.float32,
                         precision=lax.Precision.HIGHEST)


def _mm2_kernel(p_ref, w_ref, o_ref):
    a = p_ref[0] + p_ref[1]
    o_ref[...] = jnp.dot(a, w_ref[...], preferred_element_type=jnp.float32,
                         precision=lax.Precision.HIGHEST)


def _lin_kernel(xwt_ref, p_ref, wb_ref, b_ref, o_ref):
    acc = xwt_ref[...] + jnp.dot(p_ref[0] + p_ref[1], wb_ref[...],
                                 preferred_element_type=jnp.float32,
                                 precision=lax.Precision.HIGHEST)
    o_ref[...] = acc + b_ref[...]


_B = 1000
_G = N // _B


def _mm(x, w):
    return pl.pallas_call(
        _mm_kernel,
        grid=(_G,),
        in_specs=[pl.BlockSpec((_B, D), lambda i: (i, 0)),
                  pl.BlockSpec((D, D), lambda i: (0, 0))],
        out_specs=pl.BlockSpec((_B, D), lambda i: (i, 0)),
        out_shape=jax.ShapeDtypeStruct((N, D), jnp.float32),
    )(x, w)


def _mm2(parts, w):
    return pl.pallas_call(
        _mm2_kernel,
        grid=(_G,),
        in_specs=[pl.BlockSpec((NC, _B, D), lambda i: (0, i, 0)),
                  pl.BlockSpec((D, D), lambda i: (0, 0))],
        out_specs=pl.BlockSpec((_B, D), lambda i: (i, 0)),
        out_shape=jax.ShapeDtypeStruct((N, D), jnp.float32),
    )(parts, w)


def _lin(xwt, parts, wb, b):
    return pl.pallas_call(
        _lin_kernel,
        grid=(_G,),
        in_specs=[pl.BlockSpec((_B, D), lambda i: (i, 0)),
                  pl.BlockSpec((NC, _B, D), lambda i: (0, i, 0)),
                  pl.BlockSpec((D, D), lambda i: (0, 0)),
                  pl.BlockSpec((1, D), lambda i: (0, 0))],
        out_specs=pl.BlockSpec((_B, D), lambda i: (i, 0)),
        out_shape=jax.ShapeDtypeStruct((N, D), jnp.float32),
    )(xwt, parts, wb, b)


def kernel(x, edge_index, edge_weight, W_v2e, W_e2v, W_lin, b_lin):
    src = edge_index[0]
    dst = edge_index[1]
    pad = E_PAD - E
    pad_idx = (jnp.arange(pad, dtype=jnp.int32) * 97) % N
    shp = (E_PAD // (8 * C), 8, C)
    srcp = jnp.concatenate([src, pad_idx]).reshape(shp)
    dstp = jnp.concatenate([dst, pad_idx]).reshape(shp)
    wp = jnp.concatenate(
        [edge_weight, jnp.zeros((pad,), jnp.float32)]).reshape(shp)

    h1 = _mm(x, W_v2e)
    xwt = _mm(x, W_lin[:D])
    xe_parts = _sc_phase(h1, srcp, dstp, wp)
    h2 = _mm2(xe_parts, W_e2v)
    xv_parts = _sc_phase(h2, dstp, srcp, wp)
    return _lin(xwt, xv_parts, W_lin[D:], b_lin[None, :])
